# baseline (speedup 1.0000x reference)
import functools
import math

import jax
import jax.numpy as jnp
from jax.experimental import pallas as pl

N = 8192
DIN = 128
RATIO = 0.5


def _linear_kernel(x_ref, w_ref, b_ref, o_ref):
    o_ref[...] = jnp.dot(x_ref[...], w_ref[...],
                         preferred_element_type=jnp.float32) + b_ref[...]


def _linear(x, W, b):
    n = x.shape[0]
    bm = min(n, 1024)
    return pl.pallas_call(
        _linear_kernel,
        grid=(n // bm,),
        in_specs=[
            pl.BlockSpec((bm, x.shape[1]), lambda i: (i, 0)),
            pl.BlockSpec((x.shape[1], W.shape[1]), lambda i: (0, 0)),
            pl.BlockSpec((1, W.shape[1]), lambda i: (0, 0)),
        ],
        out_specs=pl.BlockSpec((bm, W.shape[1]), lambda i: (i, 0)),
        out_shape=jax.ShapeDtypeStruct((n, W.shape[1]), jnp.float32),
    )(x, W, b.reshape(1, -1))


def _hmm_kernel(h_ref, y_ref, o_ref):
    acc = jnp.dot(h_ref[...], y_ref[...], preferred_element_type=jnp.float32)
    o_ref[...] = jnp.maximum(acc, 0.0)


def _hconv(H, y):
    n, d = y.shape
    m = H.shape[0]
    bm = min(m, 256)
    return pl.pallas_call(
        _hmm_kernel,
        grid=(m // bm,),
        in_specs=[
            pl.BlockSpec((bm, n), lambda i: (i, 0)),
            pl.BlockSpec((n, d), lambda i: (0, 0)),
        ],
        out_specs=pl.BlockSpec((bm, d), lambda i: (i, 0)),
        out_shape=jax.ShapeDtypeStruct((m, d), jnp.float32),
    )(H, y)


def kernel(feat, H, p0, p1, p2, Wd0, bd0, Wd1, bd1, Wd2, bd2,
           Wu0, bu0, Wu1, bu1, Wu2, bu2):
    ps = [p0, p1, p2]
    Wds = [Wd0, Wd1, Wd2]
    bds = [bd0, bd1, bd2]
    Wus = [Wu0, Wu1, Wu2]
    bus = [bu0, bu1, bu2]

    x = feat
    xsaved = [x]
    graphs = [H]
    perms = []
    Hc = H
    n = N
    for i in range(3):
        p = ps[i]
        score = (x @ p) / (jnp.linalg.norm(p) + 1e-16)
        k = int(math.ceil(RATIO * n))
        topvals, perm = jax.lax.top_k(score, k)
        x = x[perm] * jnp.tanh(topvals)[:, None]
        Hc = Hc[perm][:, perm]
        y = _linear(x, Wds[i], bds[i])
        x = _hconv(Hc, y)
        if i < 2:
            xsaved.append(x)
            graphs.append(Hc)
        perms.append(perm)
        n = k
    for i in range(3):
        j = 2 - i
        res = xsaved[j]
        Hj = graphs[j]
        perm = perms[j]
        up = jnp.zeros_like(res).at[perm].set(x)
        x = res + up
        y = _linear(x, Wus[i], bus[i])
        x = _hconv(Hj, y)
    return x
